# baseline (speedup 1.0000x reference)
import functools

import jax
import jax.numpy as jnp
from jax import lax
from jax.experimental import pallas as pl
from jax.experimental.pallas import tpu as pltpu
from jax.experimental.pallas import tpu_sc as plsc

SIZE = 8192
BATCH = 1024
ROW = 128 * 128

NC = 2
NS = 16
NW = NC * NS
BPW = BATCH // NW
CH = 2
NCHUNK = BPW // CH

_mesh = plsc.VectorSubcoreMesh(core_axis_name="c", subcore_axis_name="s")


@functools.partial(
    pl.kernel,
    mesh=_mesh,
    out_type=[
        jax.ShapeDtypeStruct((BATCH, ROW), jnp.float32),
        jax.ShapeDtypeStruct((BATCH, ROW), jnp.float32),
    ],
    scratch_types=[
        pltpu.VMEM((2, 16), jnp.int32),
        pltpu.VMEM((2, 16), jnp.int32),
        pltpu.VMEM((CH, ROW), jnp.float32),
        pltpu.SemaphoreType.DMA,
    ],
)
def _sc_gather(state_hbm, keys_hbm, oprev_hbm, ocur_hbm,
               keys_v, km1_v, buf, sem):
    wid = lax.axis_index("s") * NC + lax.axis_index("c")
    pltpu.sync_copy(keys_hbm.at[pl.ds(wid * 2, 2)], keys_v)
    for j in range(2):
        km1_v[j] = keys_v[j] - 1
    for c in range(NCHUNK):
        j, off = divmod(c, 16 // CH)
        obase = wid * BPW + c * CH
        idx_cur = keys_v.at[j, pl.ds(off * CH, CH)]
        idx_prev = km1_v.at[j, pl.ds(off * CH, CH)]
        pltpu.sync_copy(state_hbm.at[idx_cur], buf)
        pltpu.sync_copy(buf, ocur_hbm.at[pl.ds(obase, CH)])
        pltpu.sync_copy(state_hbm.at[idx_prev], buf)
        pltpu.sync_copy(buf, oprev_hbm.at[pl.ds(obase, CH)])


def _tc_small_body(keys_ref, a_ref, r_ref, oa_ref, or_ref):
    i = pl.program_id(0)

    @pl.when(i == 0)
    def _():
        oa_ref[...] = jnp.zeros_like(oa_ref)
        or_ref[...] = jnp.zeros_like(or_ref)

    k = keys_ref[...]
    ids = lax.broadcasted_iota(jnp.int32, (BATCH, 1024), 1) + i * 1024
    oh = (k == ids).astype(jnp.float32)
    oa_ref[...] += jnp.dot(oh, a_ref[...], preferred_element_type=jnp.float32)
    or_ref[...] += jnp.dot(oh, r_ref[...], preferred_element_type=jnp.float32)


_tc_small = pl.pallas_call(
    _tc_small_body,
    grid=(SIZE // 1024,),
    in_specs=[
        pl.BlockSpec((BATCH, 1), lambda i: (0, 0)),
        pl.BlockSpec((1024, 4), lambda i: (i, 0)),
        pl.BlockSpec((1024, 1), lambda i: (i, 0)),
    ],
    out_specs=[
        pl.BlockSpec((BATCH, 4), lambda i: (0, 0)),
        pl.BlockSpec((BATCH, 1), lambda i: (0, 0)),
    ],
    out_shape=[
        jax.ShapeDtypeStruct((BATCH, 4), jnp.float32),
        jax.ShapeDtypeStruct((BATCH, 1), jnp.float32),
    ],
)


@jax.jit
def kernel(state, action, reward, keys):
    state2d = state.reshape(SIZE, ROW)
    keys_i32 = keys.astype(jnp.int32)
    s_prev, s_cur = _sc_gather(state2d, keys_i32.reshape(64, 16))
    a, r = _tc_small(keys_i32.reshape(BATCH, 1), action, reward)
    out_shape = (BATCH,) + state.shape[1:]
    return (s_prev.reshape(out_shape), a, s_cur.reshape(out_shape), r)


# speedup vs baseline: 1.0371x; 1.0371x over previous
import functools

import jax
import jax.numpy as jnp
from jax import lax
from jax.experimental import pallas as pl
from jax.experimental.pallas import tpu as pltpu
from jax.experimental.pallas import tpu_sc as plsc

SIZE = 8192
BATCH = 1024
ROW = 128 * 128

NC = 2
NS = 16
NW = NC * NS
BPW = BATCH // NW
RING = 6
DEPTH = 2

_mesh = plsc.VectorSubcoreMesh(core_axis_name="c", subcore_axis_name="s")


@functools.partial(
    pl.kernel,
    mesh=_mesh,
    out_type=[
        jax.ShapeDtypeStruct((BATCH, ROW), jnp.float32),
        jax.ShapeDtypeStruct((BATCH, ROW), jnp.float32),
    ],
    scratch_types=[
        pltpu.VMEM((2, 16), jnp.int32),
        pltpu.VMEM((2, 16), jnp.int32),
    ]
    + [pltpu.VMEM((1, ROW), jnp.float32) for _ in range(RING)]
    + [pltpu.SemaphoreType.DMA for _ in range(2 * RING)],
)
def _sc_gather(state_hbm, keys_hbm, oprev_hbm, ocur_hbm,
               keys_v, km1_v, *bufs_and_sems):
    bufs = bufs_and_sems[:RING]
    gsem = bufs_and_sems[RING:2 * RING]
    ssem = bufs_and_sems[2 * RING:]
    wid = lax.axis_index("s") * NC + lax.axis_index("c")
    pltpu.sync_copy(keys_hbm.at[pl.ds(wid * 2, 2)], keys_v)
    for j in range(2):
        km1_v[j] = keys_v[j] - 1

    items = []
    for c in range(BPW):
        j, off = divmod(c, 16)
        items.append((keys_v.at[j, pl.ds(off, 1)], ocur_hbm, c))
        items.append((km1_v.at[j, pl.ds(off, 1)], oprev_hbm, c))
    n = len(items)

    hg = [None] * n
    hs = [None] * n

    def g_start(i):
        idx, _, _ = items[i]
        b = i % RING
        hg[i] = pltpu.async_copy(state_hbm.at[idx], bufs[b], gsem[b])

    def s_start(i):
        _, out, c = items[i]
        b = i % RING
        hs[i] = pltpu.async_copy(bufs[b], out.at[pl.ds(wid * BPW + c, 1)],
                                 ssem[b])

    for i in range(DEPTH):
        g_start(i)
    for k in range(n):
        gi = k + DEPTH
        if gi < n:
            si = gi - RING
            if si >= 0:
                hs[si].wait()
            g_start(gi)
        hg[k].wait()
        s_start(k)
    for i in range(n - RING, n):
        hs[i].wait()


def _tc_small_body(keys_ref, a_ref, r_ref, oa_ref, or_ref):
    i = pl.program_id(0)

    @pl.when(i == 0)
    def _():
        oa_ref[...] = jnp.zeros_like(oa_ref)
        or_ref[...] = jnp.zeros_like(or_ref)

    k = keys_ref[...]
    ids = lax.broadcasted_iota(jnp.int32, (BATCH, 1024), 1) + i * 1024
    oh = (k == ids).astype(jnp.float32)
    oa_ref[...] += jnp.dot(oh, a_ref[...], preferred_element_type=jnp.float32,
                           precision=lax.Precision.HIGHEST)
    or_ref[...] += jnp.dot(oh, r_ref[...], preferred_element_type=jnp.float32,
                           precision=lax.Precision.HIGHEST)


_tc_small = pl.pallas_call(
    _tc_small_body,
    grid=(SIZE // 1024,),
    in_specs=[
        pl.BlockSpec((BATCH, 1), lambda i: (0, 0)),
        pl.BlockSpec((1024, 4), lambda i: (i, 0)),
        pl.BlockSpec((1024, 1), lambda i: (i, 0)),
    ],
    out_specs=[
        pl.BlockSpec((BATCH, 4), lambda i: (0, 0)),
        pl.BlockSpec((BATCH, 1), lambda i: (0, 0)),
    ],
    out_shape=[
        jax.ShapeDtypeStruct((BATCH, 4), jnp.float32),
        jax.ShapeDtypeStruct((BATCH, 1), jnp.float32),
    ],
)


@jax.jit
def kernel(state, action, reward, keys):
    state2d = state.reshape(SIZE, ROW)
    keys_i32 = keys.astype(jnp.int32)
    s_prev, s_cur = _sc_gather(state2d, keys_i32.reshape(64, 16))
    a, r = _tc_small(keys_i32.reshape(BATCH, 1), action, reward)
    out_shape = (BATCH,) + state.shape[1:]
    return (s_prev.reshape(out_shape), a, s_cur.reshape(out_shape), r)


# speedup vs baseline: 5.2426x; 5.0549x over previous
import functools

import jax
import jax.numpy as jnp
from jax import lax
from jax.experimental import pallas as pl
from jax.experimental.pallas import tpu as pltpu
from jax.experimental.pallas import tpu_sc as plsc

SIZE = 8192
BATCH = 1024
R0, R1 = 128, 128

NC = 2
NS = 16
NW = NC * NS
BPW = BATCH // NW
RING = 6
DEPTH = 2

_mesh = plsc.VectorSubcoreMesh(core_axis_name="c", subcore_axis_name="s")


@functools.partial(
    pl.kernel,
    mesh=_mesh,
    out_type=[
        jax.ShapeDtypeStruct((BATCH, R0, R1), jnp.float32),
        jax.ShapeDtypeStruct((BATCH, R0, R1), jnp.float32),
    ],
    scratch_types=[
        pltpu.VMEM((2, 16), jnp.int32),
        pltpu.VMEM((2, 16), jnp.int32),
    ]
    + [pltpu.VMEM((1, R0, R1), jnp.float32) for _ in range(RING)]
    + [pltpu.SemaphoreType.DMA for _ in range(2 * RING)],
)
def _sc_gather(state_hbm, keys_hbm, oprev_hbm, ocur_hbm,
               keys_v, km1_v, *bufs_and_sems):
    bufs = bufs_and_sems[:RING]
    gsem = bufs_and_sems[RING:2 * RING]
    ssem = bufs_and_sems[2 * RING:]
    wid = lax.axis_index("s") * NC + lax.axis_index("c")
    pltpu.sync_copy(keys_hbm.at[pl.ds(wid * 2, 2)], keys_v)
    for j in range(2):
        km1_v[j] = keys_v[j] - 1

    items = []
    for c in range(BPW):
        j, off = divmod(c, 16)
        items.append((keys_v.at[j, pl.ds(off, 1)], ocur_hbm, c))
        items.append((km1_v.at[j, pl.ds(off, 1)], oprev_hbm, c))
    n = len(items)

    hg = [None] * n
    hs = [None] * n

    def g_start(i):
        idx, _, _ = items[i]
        b = i % RING
        hg[i] = pltpu.async_copy(state_hbm.at[idx], bufs[b], gsem[b])

    def s_start(i):
        _, out, c = items[i]
        b = i % RING
        hs[i] = pltpu.async_copy(bufs[b], out.at[pl.ds(wid * BPW + c, 1)],
                                 ssem[b])

    for i in range(DEPTH):
        g_start(i)
    for k in range(n):
        gi = k + DEPTH
        if gi < n:
            si = gi - RING
            if si >= 0:
                hs[si].wait()
            g_start(gi)
        hg[k].wait()
        s_start(k)
    for i in range(n - RING, n):
        hs[i].wait()


def _tc_small_body(keys_ref, a_ref, r_ref, oa_ref, or_ref):
    i = pl.program_id(0)

    @pl.when(i == 0)
    def _():
        oa_ref[...] = jnp.zeros_like(oa_ref)
        or_ref[...] = jnp.zeros_like(or_ref)

    k = keys_ref[...]
    ids = lax.broadcasted_iota(jnp.int32, (BATCH, 1024), 1) + i * 1024
    oh = (k == ids).astype(jnp.float32)
    oa_ref[...] += jnp.dot(oh, a_ref[...], preferred_element_type=jnp.float32,
                           precision=lax.Precision.HIGHEST)
    or_ref[...] += jnp.dot(oh, r_ref[...], preferred_element_type=jnp.float32,
                           precision=lax.Precision.HIGHEST)


_tc_small = pl.pallas_call(
    _tc_small_body,
    grid=(SIZE // 1024,),
    in_specs=[
        pl.BlockSpec((BATCH, 1), lambda i: (0, 0)),
        pl.BlockSpec((1024, 4), lambda i: (i, 0)),
        pl.BlockSpec((1024, 1), lambda i: (i, 0)),
    ],
    out_specs=[
        pl.BlockSpec((BATCH, 4), lambda i: (0, 0)),
        pl.BlockSpec((BATCH, 1), lambda i: (0, 0)),
    ],
    out_shape=[
        jax.ShapeDtypeStruct((BATCH, 4), jnp.float32),
        jax.ShapeDtypeStruct((BATCH, 1), jnp.float32),
    ],
)


@jax.jit
def kernel(state, action, reward, keys):
    state3d = state.reshape(SIZE, R0, R1)
    keys_i32 = keys.astype(jnp.int32)
    s_prev, s_cur = _sc_gather(state3d, keys_i32.reshape(64, 16))
    a, r = _tc_small(keys_i32.reshape(BATCH, 1), action, reward)
    out_shape = (BATCH,) + state.shape[1:]
    return (s_prev.reshape(out_shape), a, s_cur.reshape(out_shape), r)
